# baseline (speedup 1.0000x reference)
import jax
import jax.numpy as jnp
from jax.experimental import pallas as pl
from jax.experimental.pallas import tpu as pltpu
from functools import partial

_NC = 80
_NA = 3
_EPS = 1e-5
_VMEM = 56 * 1024 * 1024

_INTERPRET = False


_BUDGET = 40 * 1024 * 1024


def _pick_blocks(m, k, c):
    bms = [b for b in (2704, 1352) if b < m and m % b == 0]
    if m <= 2704:
        bms = [m] + bms
    if k <= 1216:
        bks = [k]
    else:
        bks = [d for d in (1152, 768, 512, 384, 256, 128) if k % d == 0]
    for bm in bms:
        for bk in bks:
            if (bm * bk + bk * c + bm * c) * 16 <= _BUDGET:
                return bm, bk
    return bms[-1], bks[-1]


def _pick_bm_elem(m, c, nops):
    bms = [b for b in (2704, 1352) if b < m and m % b == 0]
    if m <= 2704:
        bms = [m] + bms
    for bm in bms:
        if bm * c * 8 * nops <= _BUDGET:
            return bm
    return bms[-1]


def _mm_stats(x2, w2, bias=None, want_stats=True, bk_override=None):
    m, k = x2.shape
    c = w2.shape[1]
    if k > 1024:
        kp = ((k + 255) // 256) * 256
        if kp != k:
            x2 = jnp.pad(x2, ((0, 0), (0, kp - k)))
            w2 = jnp.pad(w2, ((0, kp - k), (0, 0)))
            k = kp
    bm, bk = _pick_blocks(m, k, c)
    if bk_override is not None:
        bk = bk_override
    if k > 1024:
        bk = 256
        bm, _ = _pick_blocks(m, bk, c)
    gm, gk = m // bm, k // bk

    def kern(*refs):
        if bias is not None:
            x_ref, w_ref, b_ref = refs[:3]
            rest = refs[3:]
        else:
            x_ref, w_ref = refs[:2]
            rest = refs[2:]
        if want_stats:
            y_ref, st_ref = rest
        else:
            (y_ref,) = rest
        ki = pl.program_id(1)

        @pl.when(ki == 0)
        def _():
            y_ref[...] = jnp.zeros_like(y_ref)

        y_ref[...] += jnp.dot(x_ref[...], w_ref[...],
                              preferred_element_type=jnp.float32)

        @pl.when(ki == gk - 1)
        def _():
            if bias is not None:
                y_ref[...] += b_ref[...]
            if want_stats:
                a = y_ref[...]
                st_ref[0, 0:1, :] = jnp.sum(a, axis=0, keepdims=True)
                st_ref[0, 1:2, :] = jnp.sum(a * a, axis=0, keepdims=True)

    in_specs = [
        pl.BlockSpec((bm, bk), lambda mi, ki: (mi, ki)),
        pl.BlockSpec((bk, c), lambda mi, ki: (ki, 0)),
    ]
    inputs = [x2, w2]
    if bias is not None:
        in_specs.append(pl.BlockSpec((1, c), lambda mi, ki: (0, 0)))
        inputs.append(bias.reshape(1, c))

    out_shape = [jax.ShapeDtypeStruct((m, c), jnp.float32)]
    out_specs = [pl.BlockSpec((bm, c), lambda mi, ki: (mi, 0))]
    if want_stats:
        out_shape.append(jax.ShapeDtypeStruct((gm, 8, c), jnp.float32))
        out_specs.append(pl.BlockSpec((1, 8, c), lambda mi, ki: (mi, 0, 0)))

    res = pl.pallas_call(
        kern,
        grid=(gm, gk),
        in_specs=in_specs,
        out_specs=out_specs,
        out_shape=out_shape,
        compiler_params=pltpu.CompilerParams(
            dimension_semantics=("parallel", "arbitrary"),
            vmem_limit_bytes=_VMEM,
        ),
        name=f"mm_{m}x{k}x{c}",
        interpret=_INTERPRET,
    )(*inputs)
    if want_stats:
        y, st = res
        return y, st[:, 0, :].sum(0), st[:, 1, :].sum(0)
    return res[0]


def _finalize(y, scale, shift, res=None):
    m, c = y.shape
    bm = _pick_bm_elem(m, c, 4 if res is not None else 3)
    gm = m // bm

    def kern(*refs):
        if res is not None:
            y_ref, sc_ref, sh_ref, r_ref, o_ref = refs
        else:
            y_ref, sc_ref, sh_ref, o_ref = refs
        v = jnp.maximum(y_ref[...] * sc_ref[...] + sh_ref[...], 0.0)
        if res is not None:
            v = v + r_ref[...]
        o_ref[...] = v

    in_specs = [
        pl.BlockSpec((bm, c), lambda mi: (mi, 0)),
        pl.BlockSpec((1, c), lambda mi: (0, 0)),
        pl.BlockSpec((1, c), lambda mi: (0, 0)),
    ]
    inputs = [y, scale.reshape(1, c), shift.reshape(1, c)]
    if res is not None:
        in_specs.append(pl.BlockSpec((bm, c), lambda mi: (mi, 0)))
        inputs.append(res)

    return pl.pallas_call(
        kern,
        grid=(gm,),
        in_specs=in_specs,
        out_specs=pl.BlockSpec((bm, c), lambda mi: (mi, 0)),
        out_shape=jax.ShapeDtypeStruct((m, c), jnp.float32),
        compiler_params=pltpu.CompilerParams(
            dimension_semantics=("parallel",),
            vmem_limit_bytes=_VMEM,
        ),
        name=f"fin_{m}x{c}",
        interpret=_INTERPRET,
    )(*inputs)


def _im2col(xf, stride):
    b, h, w, c = xf.shape
    ho, wo = h // stride, w // stride
    xp = jnp.pad(xf, ((0, 0), (1, 1), (1, 1), (0, 0)))
    cols = []
    for dy in range(3):
        for dx in range(3):
            cols.append(xp[:, dy:dy + (ho - 1) * stride + 1:stride,
                           dx:dx + (wo - 1) * stride + 1:stride, :])
    return jnp.concatenate(cols, axis=-1).reshape(b * ho * wo, 9 * c)


def _cbr(xf, p, stride=1, res=None):
    b, h, w, cin = xf.shape
    wt = p["w"]
    cout, ksz = wt.shape[0], wt.shape[-1]
    if ksz == 1:
        x2 = xf.reshape(b * h * w, cin)
        w2 = wt[:, :, 0, 0].T
        ho, wo = h, w
    else:
        x2 = _im2col(xf, stride)
        w2 = jnp.transpose(wt, (2, 3, 1, 0)).reshape(9 * cin, cout)
        ho, wo = h // stride, w // stride
    y, s, ss = _mm_stats(x2, w2)
    n = jnp.float32(x2.shape[0])
    mean = s / n
    var = ss / n - mean * mean
    scale = p["g"] * jax.lax.rsqrt(var + _EPS)
    shift = p["b"] - mean * scale
    r2 = None if res is None else res.reshape(-1, cout)
    out = _finalize(y, scale, shift, r2)
    return out.reshape(b, ho, wo, cout)


def _resb(xf, p):
    return _cbr(_cbr(xf, p["a"]), p["b"], res=xf)


def _up2(xf):
    return jnp.repeat(jnp.repeat(xf, 2, axis=1), 2, axis=2)


def _head(f, hp):
    b, h, w, _ = f.shape
    f = _cbr(_cbr(f, hp["c0"]), hp["c1"])
    x2 = f.reshape(b * h * w, f.shape[-1])
    w2 = hp["pw"][:, :, 0, 0].T
    y = _mm_stats(x2, w2, bias=hp["pb"], want_stats=False)
    y = y.reshape(b, h, w, _NA, 5 + _NC)
    return jnp.transpose(y, (0, 3, 1, 2, 4))


def kernel(x, params):
    xf = jnp.transpose(x, (0, 2, 3, 1))
    bb = params["backbone"]
    h = _cbr(xf, bb["s0_0"], 1)
    h = _cbr(h, bb["s0_1"], 2)
    h = _cbr(h, bb["s1_0"], 2)
    h = _resb(h, bb["s1_r0"])
    h = _cbr(h, bb["s2_0"], 2)
    h = _resb(h, bb["s2_r0"])
    p3 = _resb(h, bb["s2_r1"])
    h = _cbr(p3, bb["s3_0"], 2)
    for r in bb["s3_res"]:
        h = _resb(h, r)
    p4 = h
    h = _cbr(p4, bb["s4_0"], 2)
    h = _resb(h, bb["s4_r0"])
    p5 = _cbr(h, bb["s4_1"], 1)

    fp = params["fpn"]
    n5 = _cbr(p5, fp["lat5"], 1)
    n4 = _cbr(p4, fp["lat4"], 1, res=_up2(n5))
    n3 = _cbr(p3, fp["lat3"], 1, res=_up2(n4))
    n3 = _cbr(n3, fp["out3"], 1)
    n4 = _cbr(n4, fp["out4"], 1)
    n5 = _cbr(n5, fp["out5"], 1)

    return (_head(n3, params["head_s"]),
            _head(n4, params["head_m"]),
            _head(n5, params["head_l"]))
